# baseline (speedup 1.0000x reference)
import functools

import jax
import jax.numpy as jnp
import numpy as np
from jax import lax
from jax.experimental import pallas as pl
from jax.experimental.pallas import tpu as pltpu
from jax.experimental.pallas import tpu_sc as plsc

N = 10000
E = 320000
G = 64
HID = 32
T = 4
TH = T * HID
NB = 64
BKT = 160
NPAD = NB * BKT
NT = 32
CHUNK = E // NT
CHUNKB = 32
REG = CHUNK + NB * 8 + 2 * CHUNKB
FMAX = float(np.finfo(np.float32).max)
AVG_DEG_LOG = float(np.log(33.0))

_memo = {}


def _mesh():
    if "mesh" not in _memo:
        _memo["mesh"] = plsc.VectorSubcoreMesh(core_axis_name="c",
                                               subcore_axis_name="s")
    return _memo["mesh"]


def _lane_bcast(v, l):
    idx = jnp.full((16, 1), l, jnp.int32)
    return lax.gather(
        v, idx,
        lax.GatherDimensionNumbers(offset_dims=(), collapsed_slice_dims=(0,),
                                   start_index_map=(0,)),
        (1,), mode=lax.GatherScatterMode.PROMISE_IN_BOUNDS)


def _bucketize_kernel():
    if "pa" in _memo:
        return _memo["pa"]

    @functools.partial(
        pl.kernel,
        out_type=(jax.ShapeDtypeStruct((NT * REG,), jnp.int32),
                  jax.ShapeDtypeStruct((NT * 128,), jnp.int32)),
        mesh=_mesh(),
        compiler_params=pltpu.CompilerParams(needs_layout_passes=False),
        scratch_types=[pltpu.VMEM((CHUNK,), jnp.int32),
                       pltpu.VMEM((CHUNK,), jnp.int32),
                       pltpu.VMEM((REG,), jnp.int32),
                       pltpu.VMEM((128,), jnp.int32)])
    def pa(src_hbm, dst_hbm, lists_hbm, tab_hbm, srcv, dstv, stage, tabv):
        wid = lax.axis_index("s") * 2 + lax.axis_index("c")
        base = wid * CHUNK
        pltpu.sync_copy(src_hbm.at[pl.ds(base, CHUNK)], srcv)
        pltpu.sync_copy(dst_hbm.at[pl.ds(base, CHUNK)], dstv)
        iot = lax.iota(jnp.int32, 16)

        def pre(i, c):
            sl = pl.ds(i * 16, 16)
            srcv[sl] = srcv[sl] << 8
            return c
        lax.fori_loop(0, CHUNK // 16, pre, 0)

        zero = jnp.zeros((16,), jnp.int32)

        def bucket(b, carry):
            ptr, o0, o1, o2, o3, c0, c1, c2, c3 = carry
            lo = b * BKT

            def vec(i, pv):
                sl = pl.ds(i * 16, 16)
                dl = dstv[sl] - lo
                m = (dl >= 0) & (dl < BKT)
                packed = srcv[sl] + dl
                pos = plsc.cumsum(m.astype(jnp.int32))
                plsc.store_scatter(stage, [pv + pos - 1], packed, mask=m)
                return pv + _lane_bcast(pos, 15)

            p2 = lax.fori_loop(0, CHUNK // 16, vec, ptr)
            cnt = p2 - ptr
            p3 = (p2 + 7) & (-8)
            g = b // 16
            l = b - g * 16
            sel = iot == l
            o0 = jnp.where(sel & (g == 0), ptr, o0)
            o1 = jnp.where(sel & (g == 1), ptr, o1)
            o2 = jnp.where(sel & (g == 2), ptr, o2)
            o3 = jnp.where(sel & (g == 3), ptr, o3)
            c0 = jnp.where(sel & (g == 0), cnt, c0)
            c1 = jnp.where(sel & (g == 1), cnt, c1)
            c2 = jnp.where(sel & (g == 2), cnt, c2)
            c3 = jnp.where(sel & (g == 3), cnt, c3)
            return p3, o0, o1, o2, o3, c0, c1, c2, c3

        res = lax.fori_loop(0, NB, bucket, (zero,) * 9)
        for g in range(4):
            tabv[pl.ds(g * 16, 16)] = res[1 + g]
            tabv[pl.ds(64 + g * 16, 16)] = res[5 + g]
        pltpu.sync_copy(stage, lists_hbm.at[pl.ds(wid * REG, REG)])
        pltpu.sync_copy(tabv, tab_hbm.at[pl.ds(wid * 128, 128)])

    _memo["pa"] = pa
    return pa


def _extract(tabbuf, a, row, b, iot):
    g16 = (b // 16) * 16
    l = b - g16
    v = tabbuf[pl.ds(a * 128 + row * 64 + g16, 16)]
    return jnp.sum(jnp.where(iot == l, v, 0))


def _stats_kernel():
    if "pb" in _memo:
        return _memo["pb"]

    fstruct = jax.ShapeDtypeStruct((NPAD * TH,), jnp.float32)

    @functools.partial(
        pl.kernel,
        out_type=(fstruct, fstruct, fstruct, fstruct,
                  jax.ShapeDtypeStruct((NPAD,), jnp.float32)),
        mesh=_mesh(),
        compiler_params=pltpu.CompilerParams(needs_layout_passes=False),
        scratch_types=[pltpu.VMEM((CHUNKB,), jnp.int32),
                       pltpu.VMEM((CHUNKB,), jnp.int32),
                       pltpu.VMEM((CHUNKB,), jnp.int32),
                       pltpu.VMEM((CHUNKB,), jnp.int32),
                       pltpu.VMEM((CHUNKB, TH), jnp.float32),
                       pltpu.VMEM((CHUNKB, TH), jnp.float32),
                       pltpu.VMEM((BKT * TH,), jnp.float32),
                       pltpu.VMEM((BKT * TH,), jnp.float32),
                       pltpu.VMEM((BKT * TH,), jnp.float32),
                       pltpu.VMEM((BKT * TH,), jnp.float32),
                       pltpu.VMEM((BKT,), jnp.float32),
                       pltpu.VMEM((NT * 128,), jnp.int32),
                       pltpu.SemaphoreType.DMA,
                       pltpu.SemaphoreType.DMA,
                       pltpu.SemaphoreType.DMA,
                       pltpu.SemaphoreType.DMA])
    def pb(bt_hbm, lists_hbm, tab_hbm,
           s1_hbm, s2_hbm, mn_hbm, mx_hbm, dg_hbm,
           ebuf0, ebuf1, idx0, idx1, rows0, rows1,
           a1, a2, amn, amx, dacc, tabbuf, semL0, semL1, semG0, semG1):
        wid = lax.axis_index("s") * 2 + lax.axis_index("c")
        iot = lax.iota(jnp.int32, 16)
        pltpu.sync_copy(tab_hbm, tabbuf)
        zf = jnp.zeros((16,), jnp.float32)
        onesf = jnp.ones((16,), jnp.float32)
        vmax = jnp.full((16,), FMAX, jnp.float32)

        def build_idx(ebuf, idxbuf):
            for g in range(CHUNKB // 16):
                sl = pl.ds(g * 16, 16)
                s = ebuf[sl] >> 8
                idxbuf[sl] = jnp.minimum(jnp.maximum(s, 0), N - 1)

        def process(ebuf, rows, rem):
            for g in range(CHUNKB // 16):
                dl = ebuf[pl.ds(g * 16, 16)] & 255
                for l in range(16):
                    e = g * 16 + l

                    @pl.when(e < rem)
                    def _edge(dl=dl, l=l, e=e):
                        dls = dl[l]
                        base = dls * TH

                        for joff in range(0, TH, 16):
                            sl = pl.ds(base + joff, 16)
                            r = rows[e, pl.ds(joff, 16)]
                            plsc.addupdate(a1.at[sl], r)
                            plsc.addupdate(a2.at[sl], r * r)
                            amn[sl] = jnp.minimum(amn[sl], r)
                            amx[sl] = jnp.maximum(amx[sl], r)
                        dlb = _lane_bcast(dl, l)
                        plsc.addupdate_scatter(
                            dacc, [dlb], onesf, mask=iot == 0)

        for bb in range(2):
            b = wid * 2 + bb

            def initr(r, c):
                sl = pl.ds(r * 16, 16)
                a1[sl] = zf
                a2[sl] = zf
                amn[sl] = vmax
                amx[sl] = -vmax
                return c
            lax.fori_loop(0, BKT * TH // 16, initr, 0)
            for j in range(BKT // 16):
                dacc[pl.ds(j * 16, 16)] = zf

            def per_a(a, c):
                off = pl.multiple_of(_extract(tabbuf, a, 0, b, iot), 8)
                cnt = _extract(tabbuf, a, 1, b, iot)
                nch = (cnt + CHUNKB - 1) // CHUNKB
                npair = (nch + 1) // 2

                def per_pair(q, cc):
                    ch0 = q * 2
                    lo0 = pl.multiple_of(a * REG + off + ch0 * CHUNKB, 8)
                    lo1 = pl.multiple_of(lo0 + CHUNKB, 8)
                    cl0 = pltpu.async_copy(
                        lists_hbm.at[pl.ds(lo0, CHUNKB)], ebuf0, semL0)
                    cl1 = pltpu.async_copy(
                        lists_hbm.at[pl.ds(lo1, CHUNKB)], ebuf1, semL1)
                    cl0.wait()
                    build_idx(ebuf0, idx0)
                    g0 = pltpu.async_copy(bt_hbm.at[idx0], rows0, semG0)
                    cl1.wait()
                    build_idx(ebuf1, idx1)
                    g1 = pltpu.async_copy(bt_hbm.at[idx1], rows1, semG1)
                    g0.wait()
                    process(ebuf0, rows0, cnt - ch0 * CHUNKB)
                    g1.wait()
                    process(ebuf1, rows1, cnt - (ch0 + 1) * CHUNKB)
                    return cc
                lax.fori_loop(0, npair, per_pair, 0)
                return c
            lax.fori_loop(0, NT, per_a, 0)

            pltpu.sync_copy(a1, s1_hbm.at[pl.ds(b * BKT * TH, BKT * TH)])
            pltpu.sync_copy(a2, s2_hbm.at[pl.ds(b * BKT * TH, BKT * TH)])
            pltpu.sync_copy(amn, mn_hbm.at[pl.ds(b * BKT * TH, BKT * TH)])
            pltpu.sync_copy(amx, mx_hbm.at[pl.ds(b * BKT * TH, BKT * TH)])
            pltpu.sync_copy(dacc, dg_hbm.at[pl.ds(b * BKT, BKT)])

    _memo["pb"] = pb
    return pb


RB = 1000
NRB = N // RB


def _embed_call(x, W_emb, b_emb, Wtop, Wbot):
    def body(x_ref, w_ref, b_ref, wt_ref, wb_ref, h_ref, a_ref, bt_ref):
        hb = x_ref[...] @ w_ref[...] + b_ref[...]
        h_ref[...] = hb
        a_ref[...] = hb @ wt_ref[...]
        bt_ref[...] = hb @ wb_ref[...]

    full = lambda i: (0, 0)
    rowb = lambda i: (i, 0)
    with jax.default_matmul_precision("default"):
        return pl.pallas_call(
            body,
            grid=(NRB,),
            in_specs=[pl.BlockSpec((RB, 128), rowb),
                      pl.BlockSpec((128, HID), full),
                      pl.BlockSpec((1, HID), full),
                      pl.BlockSpec((HID, TH), full),
                      pl.BlockSpec((HID, TH), full)],
            out_specs=[pl.BlockSpec((RB, HID), rowb),
                       pl.BlockSpec((RB, TH), rowb),
                       pl.BlockSpec((RB, TH), rowb)],
            out_shape=[jax.ShapeDtypeStruct((N, HID), jnp.float32),
                       jax.ShapeDtypeStruct((N, TH), jnp.float32),
                       jax.ShapeDtypeStruct((N, TH), jnp.float32)],
        )(x, W_emb, b_emb.reshape(1, HID), Wtop, Wbot)


def _combine_call(h, A, S1, S2, Mn, Mx, degp, bias, postWc, linW, linB,
                  relu, nextw):
    Wh, Wid, Wamp, Watt, pbias = postWc

    def body(h_ref, a_ref, s1_ref, s2_ref, mn_ref, mx_ref, dg_ref, bias_ref,
             wh_ref, wid_ref, wamp_ref, watt_ref, pb_ref, lw_ref, lb_ref,
             *rest):
        if nextw is not None:
            wtn_ref, wbn_ref, h2_ref, an_ref, bn_ref = rest
        else:
            (h2_ref,) = rest
        deg = dg_ref[...]
        deg_c = jnp.maximum(deg, 1.0)
        has = deg > 0
        Ab = a_ref[...] + bias_ref[...]
        S1b = s1_ref[...]
        mean = (deg * Ab + S1b) / deg_c
        mean2 = (deg * Ab * Ab + 2.0 * Ab * S1b + s2_ref[...]) / deg_c
        std = jnp.sqrt(jnp.maximum(mean2 - mean * mean, 0.0) + 1e-5)
        mn = jnp.where(has, Ab + mn_ref[...], 0.0)
        mx = jnp.where(has, Ab + mx_ref[...], 0.0)
        lg = jnp.log(deg_c + 1.0)
        amp = lg / AVG_DEG_LOG
        att = AVG_DEG_LOG / lg
        agg = jnp.concatenate([mean, mn, mx, std], axis=-1)
        out = (h_ref[...] @ wh_ref[...] + agg @ wid_ref[...]
               + amp * (agg @ wamp_ref[...]) + att * (agg @ watt_ref[...])
               + pb_ref[...])
        h2 = out @ lw_ref[...] + lb_ref[...]
        if relu:
            h2 = jnp.maximum(h2, 0.0)
        h2_ref[...] = h2
        if nextw is not None:
            an_ref[...] = h2 @ wtn_ref[...]
            bn_ref[...] = h2 @ wbn_ref[...]

    full = lambda i: (0, 0)
    rowb = lambda i: (i, 0)
    in_specs = [pl.BlockSpec((RB, HID), rowb),
                pl.BlockSpec((RB, TH), rowb),
                pl.BlockSpec((RB, TH), rowb),
                pl.BlockSpec((RB, TH), rowb),
                pl.BlockSpec((RB, TH), rowb),
                pl.BlockSpec((RB, TH), rowb),
                pl.BlockSpec((RB, 1), rowb),
                pl.BlockSpec((1, TH), full),
                pl.BlockSpec((HID, HID), full),
                pl.BlockSpec((4 * TH, HID), full),
                pl.BlockSpec((4 * TH, HID), full),
                pl.BlockSpec((4 * TH, HID), full),
                pl.BlockSpec((1, HID), full),
                pl.BlockSpec((HID, HID), full),
                pl.BlockSpec((1, HID), full)]
    out_specs = [pl.BlockSpec((RB, HID), rowb)]
    out_shape = [jax.ShapeDtypeStruct((N, HID), jnp.float32)]
    args = [h, A, S1, S2, Mn, Mx, degp, bias.reshape(1, TH),
            Wh, Wid, Wamp, Watt, pbias.reshape(1, HID), linW,
            linB.reshape(1, HID)]
    if nextw is not None:
        in_specs += [pl.BlockSpec((HID, TH), full),
                     pl.BlockSpec((HID, TH), full)]
        out_specs += [pl.BlockSpec((RB, TH), rowb),
                      pl.BlockSpec((RB, TH), rowb)]
        out_shape += [jax.ShapeDtypeStruct((N, TH), jnp.float32),
                      jax.ShapeDtypeStruct((N, TH), jnp.float32)]
        args += [nextw[0], nextw[1]]
    with jax.default_matmul_precision("default"):
        res = pl.pallas_call(
            body, grid=(NRB,), in_specs=in_specs, out_specs=out_specs,
            out_shape=out_shape)(*args)
    return res


def _pool_call(h, batch, W1, b1, W2, b2):
    def body(h_ref, b_ref, w1_ref, b1_ref, w2_ref, b2_ref, out_ref, acc):
        i = pl.program_id(0)

        @pl.when(i == 0)
        def _init():
            acc[...] = jnp.zeros((G, HID), jnp.float32)

        iota_g = lax.broadcasted_iota(jnp.int32, (G, RB), 0)
        onehot = (b_ref[...][0] == iota_g).astype(jnp.float32)
        acc[...] += onehot @ h_ref[...]

        @pl.when(i == NRB - 1)
        def _fin():
            p = acc[...]
            hmid = jnp.maximum(p @ w1_ref[...] + b1_ref[...], 0.0)
            out_ref[...] = hmid @ w2_ref[...] + b2_ref[...]

    full = lambda i: (0, 0)
    with jax.default_matmul_precision("default"):
        return pl.pallas_call(
            body,
            grid=(NRB,),
            in_specs=[pl.BlockSpec((RB, HID), lambda i: (i, 0)),
                      pl.BlockSpec((1, 1, RB), lambda i: (i, 0, 0)),
                      pl.BlockSpec((HID, HID // 2), full),
                      pl.BlockSpec((1, HID // 2), full),
                      pl.BlockSpec((HID // 2, 2), full),
                      pl.BlockSpec((1, 2), full)],
            out_specs=pl.BlockSpec((G, 2), full),
            out_shape=jax.ShapeDtypeStruct((G, 2), jnp.float32),
            scratch_shapes=[pltpu.VMEM((G, HID), jnp.float32)],
        )(h, batch.reshape(NRB, 1, RB), W1, b1.reshape(1, HID // 2), W2,
          b2.reshape(1, 2))


def _prep_conv(preW, preB, postW, postB):
    Wtop = jnp.concatenate([preW[t][:HID] for t in range(T)], axis=1)
    Wbot = jnp.concatenate([preW[t][HID:] for t in range(T)], axis=1)
    bias = jnp.concatenate([preB[t] for t in range(T)], axis=0)
    FOUT = postW.shape[-1]
    Wh = jnp.concatenate([postW[t][:HID] for t in range(T)], axis=1)

    def seg_matrix(offset):
        M = jnp.zeros((4 * TH, T * FOUT), jnp.float32)
        for s in range(4):
            for t in range(T):
                rows = postW[t][HID + offset + s * HID:
                                HID + offset + (s + 1) * HID]
                M = M.at[s * TH + t * HID: s * TH + (t + 1) * HID,
                         t * FOUT:(t + 1) * FOUT].set(rows)
        return M

    Wid = seg_matrix(0)
    Wamp = seg_matrix(4 * HID)
    Watt = seg_matrix(8 * HID)
    pb = jnp.concatenate([postB[t] for t in range(T)], axis=0)
    return Wtop, Wbot, bias, (Wh, Wid, Wamp, Watt, pb)


def kernel(x, edge_index, batch, W_emb, b_emb,
           c0_preW, c0_preB, c0_postW, c0_postB, c0_linW, c0_linB,
           c1_preW, c1_preB, c1_postW, c1_postB, c1_linW, c1_linB,
           W1, b1, W2, b2):
    lists, tab = _bucketize_kernel()(edge_index[0], edge_index[1])
    p0 = _prep_conv(c0_preW, c0_preB, c0_postW, c0_postB)
    p1 = _prep_conv(c1_preW, c1_preB, c1_postW, c1_postB)
    Wtop0, Wbot0, bias0, post0 = p0
    Wtop1, Wbot1, bias1, post1 = p1
    h, A, Bt = _embed_call(x, W_emb, b_emb, Wtop0, Wbot0)
    steps = [
        (bias0, post0, c0_linW, c0_linB, False, (Wtop0, Wbot0)),
        (bias0, post0, c0_linW, c0_linB, True, (Wtop1, Wbot1)),
        (bias1, post1, c1_linW, c1_linB, False, (Wtop1, Wbot1)),
        (bias1, post1, c1_linW, c1_linB, True, None),
    ]
    degp = None
    for bias, post, linW, linB, relu, nextw in steps:
        S1p, S2p, Mnp, Mxp, Dg = _stats_kernel()(Bt, lists, tab)
        if degp is None:
            degp = Dg[:N].reshape(N, 1)
        res = _combine_call(h, A, S1p.reshape(NPAD, TH)[:N],
                            S2p.reshape(NPAD, TH)[:N],
                            Mnp.reshape(NPAD, TH)[:N],
                            Mxp.reshape(NPAD, TH)[:N],
                            degp, bias, post, linW, linB, relu, nextw)
        if nextw is not None:
            h, A, Bt = res
        else:
            (h,) = res
    return _pool_call(h, batch, W1, b1, W2, b2)


# speedup vs baseline: 1.2125x; 1.2125x over previous
import functools

import jax
import jax.numpy as jnp
import numpy as np
from jax import lax
from jax.experimental import pallas as pl
from jax.experimental.pallas import tpu as pltpu
from jax.experimental.pallas import tpu_sc as plsc

N = 10000
E = 320000
G = 64
HID = 32
T = 4
TH = T * HID
NB = 64
BKT = 160
NPAD = NB * BKT
NT = 32
CHUNK = E // NT
CHUNKB = 32
REG = CHUNK + NB * 8 + 2 * CHUNKB
FMAX = float(np.finfo(np.float32).max)
AVG_DEG_LOG = float(np.log(33.0))

_memo = {}


def _mesh():
    if "mesh" not in _memo:
        _memo["mesh"] = plsc.VectorSubcoreMesh(core_axis_name="c",
                                               subcore_axis_name="s")
    return _memo["mesh"]


def _lane_bcast(v, l):
    idx = jnp.full((16, 1), l, jnp.int32)
    return lax.gather(
        v, idx,
        lax.GatherDimensionNumbers(offset_dims=(), collapsed_slice_dims=(0,),
                                   start_index_map=(0,)),
        (1,), mode=lax.GatherScatterMode.PROMISE_IN_BOUNDS)


def _bucketize_kernel():
    if "pa" in _memo:
        return _memo["pa"]

    @functools.partial(
        pl.kernel,
        out_type=(jax.ShapeDtypeStruct((NT * REG,), jnp.int32),
                  jax.ShapeDtypeStruct((NT * 128,), jnp.int32)),
        mesh=_mesh(),
        compiler_params=pltpu.CompilerParams(needs_layout_passes=False),
        scratch_types=[pltpu.VMEM((CHUNK,), jnp.int32),
                       pltpu.VMEM((CHUNK,), jnp.int32),
                       pltpu.VMEM((REG,), jnp.int32),
                       pltpu.VMEM((128,), jnp.int32)])
    def pa(src_hbm, dst_hbm, lists_hbm, tab_hbm, srcv, dstv, stage, tabv):
        wid = lax.axis_index("s") * 2 + lax.axis_index("c")
        base = wid * CHUNK
        pltpu.sync_copy(src_hbm.at[pl.ds(base, CHUNK)], srcv)
        pltpu.sync_copy(dst_hbm.at[pl.ds(base, CHUNK)], dstv)
        iot = lax.iota(jnp.int32, 16)

        def pre(i, c):
            sl = pl.ds(i * 16, 16)
            srcv[sl] = srcv[sl] << 8
            return c
        lax.fori_loop(0, CHUNK // 16, pre, 0)

        zero = jnp.zeros((16,), jnp.int32)

        def bucket(b, carry):
            ptr, o0, o1, o2, o3, c0, c1, c2, c3 = carry
            lo = b * BKT

            def vec(i, pv):
                sl = pl.ds(i * 16, 16)
                dl = dstv[sl] - lo
                m = (dl >= 0) & (dl < BKT)
                packed = srcv[sl] + dl
                pos = plsc.cumsum(m.astype(jnp.int32))
                plsc.store_scatter(stage, [pv + pos - 1], packed, mask=m)
                return pv + _lane_bcast(pos, 15)

            p2 = lax.fori_loop(0, CHUNK // 16, vec, ptr)
            cnt = p2 - ptr
            p3 = (p2 + 7) & (-8)
            g = b // 16
            l = b - g * 16
            sel = iot == l
            o0 = jnp.where(sel & (g == 0), ptr, o0)
            o1 = jnp.where(sel & (g == 1), ptr, o1)
            o2 = jnp.where(sel & (g == 2), ptr, o2)
            o3 = jnp.where(sel & (g == 3), ptr, o3)
            c0 = jnp.where(sel & (g == 0), cnt, c0)
            c1 = jnp.where(sel & (g == 1), cnt, c1)
            c2 = jnp.where(sel & (g == 2), cnt, c2)
            c3 = jnp.where(sel & (g == 3), cnt, c3)
            return p3, o0, o1, o2, o3, c0, c1, c2, c3

        res = lax.fori_loop(0, NB, bucket, (zero,) * 9)
        for g in range(4):
            tabv[pl.ds(g * 16, 16)] = res[1 + g]
            tabv[pl.ds(64 + g * 16, 16)] = res[5 + g]
        pltpu.sync_copy(stage, lists_hbm.at[pl.ds(wid * REG, REG)])
        pltpu.sync_copy(tabv, tab_hbm.at[pl.ds(wid * 128, 128)])

    _memo["pa"] = pa
    return pa


def _extract(tabbuf, a, row, b, iot):
    g16 = (b // 16) * 16
    l = b - g16
    v = tabbuf[pl.ds(a * 128 + row * 64 + g16, 16)]
    return jnp.sum(jnp.where(iot == l, v, 0))


def _stats_kernel():
    if "pb" in _memo:
        return _memo["pb"]

    fstruct = jax.ShapeDtypeStruct((NPAD * TH,), jnp.float32)

    @functools.partial(
        pl.kernel,
        out_type=(fstruct, fstruct, fstruct, fstruct,
                  jax.ShapeDtypeStruct((NPAD,), jnp.float32)),
        mesh=_mesh(),
        compiler_params=pltpu.CompilerParams(needs_layout_passes=False),
        scratch_types=[pltpu.VMEM((CHUNKB,), jnp.int32),
                       pltpu.VMEM((CHUNKB,), jnp.int32),
                       pltpu.VMEM((CHUNKB,), jnp.int32),
                       pltpu.VMEM((CHUNKB,), jnp.int32),
                       pltpu.VMEM((CHUNKB, TH), jnp.float32),
                       pltpu.VMEM((CHUNKB, TH), jnp.float32),
                       pltpu.VMEM((BKT * TH,), jnp.float32),
                       pltpu.VMEM((BKT * TH,), jnp.float32),
                       pltpu.VMEM((BKT * TH,), jnp.float32),
                       pltpu.VMEM((BKT * TH,), jnp.float32),
                       pltpu.VMEM((BKT,), jnp.float32),
                       pltpu.VMEM((NT * 128,), jnp.int32),
                       pltpu.SemaphoreType.DMA,
                       pltpu.SemaphoreType.DMA,
                       pltpu.SemaphoreType.DMA,
                       pltpu.SemaphoreType.DMA])
    def pb(bt_hbm, lists_hbm, tab_hbm,
           s1_hbm, s2_hbm, mn_hbm, mx_hbm, dg_hbm,
           ebuf0, ebuf1, idx0, idx1, rows0, rows1,
           a1, a2, amn, amx, dacc, tabbuf, semL0, semL1, semG0, semG1):
        wid = lax.axis_index("s") * 2 + lax.axis_index("c")
        iot = lax.iota(jnp.int32, 16)
        pltpu.sync_copy(tab_hbm, tabbuf)
        zf = jnp.zeros((16,), jnp.float32)
        onesf = jnp.ones((16,), jnp.float32)
        vmax = jnp.full((16,), FMAX, jnp.float32)

        def build_idx(ebuf, idxbuf):
            for g in range(CHUNKB // 16):
                sl = pl.ds(g * 16, 16)
                s = ebuf[sl] >> 8
                idxbuf[sl] = jnp.minimum(jnp.maximum(s, 0), N - 1)

        def process(ebuf, rows, rem):
            for g in range(CHUNKB // 16):
                dl = ebuf[pl.ds(g * 16, 16)] & 255
                for l in range(16):
                    e = g * 16 + l

                    @pl.when(e < rem)
                    def _edge(dl=dl, l=l, e=e):
                        dls = dl[l]
                        base = dls * TH

                        @plsc.parallel_loop(0, TH, 16, unroll=TH // 16)
                        def _j(joff):
                            sl = pl.ds(base + joff, 16)
                            r = rows[e, pl.ds(joff, 16)]
                            plsc.addupdate(a1.at[sl], r)
                            plsc.addupdate(a2.at[sl], r * r)
                            amn[sl] = jnp.minimum(amn[sl], r)
                            amx[sl] = jnp.maximum(amx[sl], r)
                        dlb = _lane_bcast(dl, l)
                        plsc.addupdate_scatter(
                            dacc, [dlb], onesf, mask=iot == 0)

        for bb in range(2):
            b = wid * 2 + bb

            def initr(r, c):
                sl = pl.ds(r * 16, 16)
                a1[sl] = zf
                a2[sl] = zf
                amn[sl] = vmax
                amx[sl] = -vmax
                return c
            lax.fori_loop(0, BKT * TH // 16, initr, 0)
            for j in range(BKT // 16):
                dacc[pl.ds(j * 16, 16)] = zf

            def per_a(a, c):
                off = pl.multiple_of(_extract(tabbuf, a, 0, b, iot), 8)
                cnt = _extract(tabbuf, a, 1, b, iot)
                nch = (cnt + CHUNKB - 1) // CHUNKB
                npair = (nch + 1) // 2

                def per_pair(q, cc):
                    ch0 = q * 2
                    lo0 = pl.multiple_of(a * REG + off + ch0 * CHUNKB, 8)
                    lo1 = pl.multiple_of(lo0 + CHUNKB, 8)
                    cl0 = pltpu.async_copy(
                        lists_hbm.at[pl.ds(lo0, CHUNKB)], ebuf0, semL0)
                    cl1 = pltpu.async_copy(
                        lists_hbm.at[pl.ds(lo1, CHUNKB)], ebuf1, semL1)
                    cl0.wait()
                    build_idx(ebuf0, idx0)
                    g0 = pltpu.async_copy(bt_hbm.at[idx0], rows0, semG0)
                    cl1.wait()
                    build_idx(ebuf1, idx1)
                    g1 = pltpu.async_copy(bt_hbm.at[idx1], rows1, semG1)
                    g0.wait()
                    process(ebuf0, rows0, cnt - ch0 * CHUNKB)
                    g1.wait()
                    process(ebuf1, rows1, cnt - (ch0 + 1) * CHUNKB)
                    return cc
                lax.fori_loop(0, npair, per_pair, 0)
                return c
            lax.fori_loop(0, NT, per_a, 0)

            pltpu.sync_copy(a1, s1_hbm.at[pl.ds(b * BKT * TH, BKT * TH)])
            pltpu.sync_copy(a2, s2_hbm.at[pl.ds(b * BKT * TH, BKT * TH)])
            pltpu.sync_copy(amn, mn_hbm.at[pl.ds(b * BKT * TH, BKT * TH)])
            pltpu.sync_copy(amx, mx_hbm.at[pl.ds(b * BKT * TH, BKT * TH)])
            pltpu.sync_copy(dacc, dg_hbm.at[pl.ds(b * BKT, BKT)])

    _memo["pb"] = pb
    return pb


RB = 1000
NRB = N // RB


def _embed_call(x, W_emb, b_emb, Wtop, Wbot):
    def body(x_ref, w_ref, b_ref, wt_ref, wb_ref, h_ref, a_ref, bt_ref):
        hb = x_ref[...] @ w_ref[...] + b_ref[...]
        h_ref[...] = hb
        a_ref[...] = hb @ wt_ref[...]
        bt_ref[...] = hb @ wb_ref[...]

    full = lambda i: (0, 0)
    rowb = lambda i: (i, 0)
    with jax.default_matmul_precision("default"):
        return pl.pallas_call(
            body,
            grid=(NRB,),
            in_specs=[pl.BlockSpec((RB, 128), rowb),
                      pl.BlockSpec((128, HID), full),
                      pl.BlockSpec((1, HID), full),
                      pl.BlockSpec((HID, TH), full),
                      pl.BlockSpec((HID, TH), full)],
            out_specs=[pl.BlockSpec((RB, HID), rowb),
                       pl.BlockSpec((RB, TH), rowb),
                       pl.BlockSpec((RB, TH), rowb)],
            out_shape=[jax.ShapeDtypeStruct((N, HID), jnp.float32),
                       jax.ShapeDtypeStruct((N, TH), jnp.float32),
                       jax.ShapeDtypeStruct((N, TH), jnp.float32)],
        )(x, W_emb, b_emb.reshape(1, HID), Wtop, Wbot)


def _combine_call(h, A, S1, S2, Mn, Mx, degp, bias, postWc, linW, linB,
                  relu, nextw):
    Wh, Wid, Wamp, Watt, pbias = postWc

    def body(h_ref, a_ref, s1_ref, s2_ref, mn_ref, mx_ref, dg_ref, bias_ref,
             wh_ref, wid_ref, wamp_ref, watt_ref, pb_ref, lw_ref, lb_ref,
             *rest):
        if nextw is not None:
            wtn_ref, wbn_ref, h2_ref, an_ref, bn_ref = rest
        else:
            (h2_ref,) = rest
        deg = dg_ref[...]
        deg_c = jnp.maximum(deg, 1.0)
        has = deg > 0
        Ab = a_ref[...] + bias_ref[...]
        S1b = s1_ref[...]
        mean = (deg * Ab + S1b) / deg_c
        mean2 = (deg * Ab * Ab + 2.0 * Ab * S1b + s2_ref[...]) / deg_c
        std = jnp.sqrt(jnp.maximum(mean2 - mean * mean, 0.0) + 1e-5)
        mn = jnp.where(has, Ab + mn_ref[...], 0.0)
        mx = jnp.where(has, Ab + mx_ref[...], 0.0)
        lg = jnp.log(deg_c + 1.0)
        amp = lg / AVG_DEG_LOG
        att = AVG_DEG_LOG / lg
        agg = jnp.concatenate([mean, mn, mx, std], axis=-1)
        out = (h_ref[...] @ wh_ref[...] + agg @ wid_ref[...]
               + amp * (agg @ wamp_ref[...]) + att * (agg @ watt_ref[...])
               + pb_ref[...])
        h2 = out @ lw_ref[...] + lb_ref[...]
        if relu:
            h2 = jnp.maximum(h2, 0.0)
        h2_ref[...] = h2
        if nextw is not None:
            an_ref[...] = h2 @ wtn_ref[...]
            bn_ref[...] = h2 @ wbn_ref[...]

    full = lambda i: (0, 0)
    rowb = lambda i: (i, 0)
    in_specs = [pl.BlockSpec((RB, HID), rowb),
                pl.BlockSpec((RB, TH), rowb),
                pl.BlockSpec((RB, TH), rowb),
                pl.BlockSpec((RB, TH), rowb),
                pl.BlockSpec((RB, TH), rowb),
                pl.BlockSpec((RB, TH), rowb),
                pl.BlockSpec((RB, 1), rowb),
                pl.BlockSpec((1, TH), full),
                pl.BlockSpec((HID, HID), full),
                pl.BlockSpec((4 * TH, HID), full),
                pl.BlockSpec((4 * TH, HID), full),
                pl.BlockSpec((4 * TH, HID), full),
                pl.BlockSpec((1, HID), full),
                pl.BlockSpec((HID, HID), full),
                pl.BlockSpec((1, HID), full)]
    out_specs = [pl.BlockSpec((RB, HID), rowb)]
    out_shape = [jax.ShapeDtypeStruct((N, HID), jnp.float32)]
    args = [h, A, S1, S2, Mn, Mx, degp, bias.reshape(1, TH),
            Wh, Wid, Wamp, Watt, pbias.reshape(1, HID), linW,
            linB.reshape(1, HID)]
    if nextw is not None:
        in_specs += [pl.BlockSpec((HID, TH), full),
                     pl.BlockSpec((HID, TH), full)]
        out_specs += [pl.BlockSpec((RB, TH), rowb),
                      pl.BlockSpec((RB, TH), rowb)]
        out_shape += [jax.ShapeDtypeStruct((N, TH), jnp.float32),
                      jax.ShapeDtypeStruct((N, TH), jnp.float32)]
        args += [nextw[0], nextw[1]]
    with jax.default_matmul_precision("default"):
        res = pl.pallas_call(
            body, grid=(NRB,), in_specs=in_specs, out_specs=out_specs,
            out_shape=out_shape)(*args)
    return res


def _pool_call(h, batch, W1, b1, W2, b2):
    def body(h_ref, b_ref, w1_ref, b1_ref, w2_ref, b2_ref, out_ref, acc):
        i = pl.program_id(0)

        @pl.when(i == 0)
        def _init():
            acc[...] = jnp.zeros((G, HID), jnp.float32)

        iota_g = lax.broadcasted_iota(jnp.int32, (G, RB), 0)
        onehot = (b_ref[...][0] == iota_g).astype(jnp.float32)
        acc[...] += onehot @ h_ref[...]

        @pl.when(i == NRB - 1)
        def _fin():
            p = acc[...]
            hmid = jnp.maximum(p @ w1_ref[...] + b1_ref[...], 0.0)
            out_ref[...] = hmid @ w2_ref[...] + b2_ref[...]

    full = lambda i: (0, 0)
    with jax.default_matmul_precision("default"):
        return pl.pallas_call(
            body,
            grid=(NRB,),
            in_specs=[pl.BlockSpec((RB, HID), lambda i: (i, 0)),
                      pl.BlockSpec((1, 1, RB), lambda i: (i, 0, 0)),
                      pl.BlockSpec((HID, HID // 2), full),
                      pl.BlockSpec((1, HID // 2), full),
                      pl.BlockSpec((HID // 2, 2), full),
                      pl.BlockSpec((1, 2), full)],
            out_specs=pl.BlockSpec((G, 2), full),
            out_shape=jax.ShapeDtypeStruct((G, 2), jnp.float32),
            scratch_shapes=[pltpu.VMEM((G, HID), jnp.float32)],
        )(h, batch.reshape(NRB, 1, RB), W1, b1.reshape(1, HID // 2), W2,
          b2.reshape(1, 2))


def _prep_conv(preW, preB, postW, postB):
    Wtop = jnp.concatenate([preW[t][:HID] for t in range(T)], axis=1)
    Wbot = jnp.concatenate([preW[t][HID:] for t in range(T)], axis=1)
    bias = jnp.concatenate([preB[t] for t in range(T)], axis=0)
    FOUT = postW.shape[-1]
    Wh = jnp.concatenate([postW[t][:HID] for t in range(T)], axis=1)

    def seg_matrix(offset):
        M = jnp.zeros((4 * TH, T * FOUT), jnp.float32)
        for s in range(4):
            for t in range(T):
                rows = postW[t][HID + offset + s * HID:
                                HID + offset + (s + 1) * HID]
                M = M.at[s * TH + t * HID: s * TH + (t + 1) * HID,
                         t * FOUT:(t + 1) * FOUT].set(rows)
        return M

    Wid = seg_matrix(0)
    Wamp = seg_matrix(4 * HID)
    Watt = seg_matrix(8 * HID)
    pb = jnp.concatenate([postB[t] for t in range(T)], axis=0)
    return Wtop, Wbot, bias, (Wh, Wid, Wamp, Watt, pb)


def kernel(x, edge_index, batch, W_emb, b_emb,
           c0_preW, c0_preB, c0_postW, c0_postB, c0_linW, c0_linB,
           c1_preW, c1_preB, c1_postW, c1_postB, c1_linW, c1_linB,
           W1, b1, W2, b2):
    lists, tab = _bucketize_kernel()(edge_index[0], edge_index[1])
    p0 = _prep_conv(c0_preW, c0_preB, c0_postW, c0_postB)
    p1 = _prep_conv(c1_preW, c1_preB, c1_postW, c1_postB)
    Wtop0, Wbot0, bias0, post0 = p0
    Wtop1, Wbot1, bias1, post1 = p1
    h, A, Bt = _embed_call(x, W_emb, b_emb, Wtop0, Wbot0)
    steps = [
        (bias0, post0, c0_linW, c0_linB, False, (Wtop0, Wbot0)),
        (bias0, post0, c0_linW, c0_linB, True, (Wtop1, Wbot1)),
        (bias1, post1, c1_linW, c1_linB, False, (Wtop1, Wbot1)),
        (bias1, post1, c1_linW, c1_linB, True, None),
    ]
    degp = None
    for bias, post, linW, linB, relu, nextw in steps:
        S1p, S2p, Mnp, Mxp, Dg = _stats_kernel()(Bt, lists, tab)
        if degp is None:
            degp = Dg[:N].reshape(N, 1)
        res = _combine_call(h, A, S1p.reshape(NPAD, TH)[:N],
                            S2p.reshape(NPAD, TH)[:N],
                            Mnp.reshape(NPAD, TH)[:N],
                            Mxp.reshape(NPAD, TH)[:N],
                            degp, bias, post, linW, linB, relu, nextw)
        if nextw is not None:
            h, A, Bt = res
        else:
            (h,) = res
    return _pool_call(h, batch, W1, b1, W2, b2)
